# baseline (speedup 1.0000x reference)
import functools

import jax
import jax.numpy as jnp
from jax import lax
from jax.experimental import pallas as pl

_GRID = 512
_LAT = 16


def _lrelu(x, s):
    return jnp.maximum(x, s * x)


def _binm(m):
    return (m > 0).astype(jnp.float32)


def _conv_body(x_ref, m_ref, w_ref, a_ref, c_ref, h_ref, st_ref, *, HW, W, N,
               Ci, Co, slope, nb):
    i = pl.program_id(0)
    s0 = i * N
    lidx = lax.broadcasted_iota(jnp.int32, (1, N), 1)

    Hb = N // W
    H = nb * Hb
    rtop = jnp.maximum(i * Hb - 1, 0) * W
    rbot = jnp.minimum(i * Hb + Hb, H - 1) * W
    top = x_ref[:, pl.ds(rtop, W)]
    top = jnp.where(i == 0, 0.0, top)
    midw = x_ref[:, pl.ds(s0, N)]
    bot = x_ref[:, pl.ds(rbot, W)]
    bot = jnp.where(i == nb - 1, 0.0, bot)
    xb = jnp.concatenate([top, midw, bot], axis=1)

    if a_ref is None:
        t = xb
        mc = _binm(m_ref[:, pl.ds(s0, N)])
    else:
        mtop = m_ref[:, pl.ds(rtop, W)]
        mtop = jnp.where(i == 0, 0.0, mtop)
        mmid = m_ref[:, pl.ds(s0, N)]
        mbot = m_ref[:, pl.ds(rbot, W)]
        mbot = jnp.where(i == nb - 1, 0.0, mbot)
        mb = _binm(jnp.concatenate([mtop, mmid, mbot], axis=1))
        av = a_ref[...]
        cv = c_ref[...]
        t = _lrelu((xb * av + cv) * mb, slope)
        mc = mb[:, W:N + W]
    xs = jnp.concatenate(
        [t[:, 0:N], t[:, W:N + W], t[:, 2 * W:N + 2 * W]], axis=0)

    accs = [
        lax.dot_general(w_ref[dc], xs, (((0,), (0,)), ((), ())),
                        preferred_element_type=jnp.float32)
        for dc in range(3)
    ]
    a0 = jnp.pad(accs[0], ((0, 0), (1, 0)))[:, :N]
    a0 = jnp.where(lidx % W == 0, 0.0, a0)
    a2 = jnp.pad(accs[2], ((0, 0), (0, 1)))[:, 1:]
    a2 = jnp.where(lidx % W == W - 1, 0.0, a2)
    h = (a0 + accs[1] + a2) * mc
    h_ref[...] = h

    s = h.sum(1, keepdims=True)
    ss = (h * h).sum(1, keepdims=True)
    mn = mc.sum(1, keepdims=True)
    upd = jnp.concatenate(
        [s, ss, jnp.broadcast_to(mn, (Co, 1)), jnp.zeros((Co, 1), jnp.float32)],
        axis=1)

    @pl.when(i == 0)
    def _():
        st_ref[...] = jnp.zeros_like(st_ref)

    st_ref[...] = st_ref[...] + upd


def _conv3x3_flat(x, m, w, a, c, slope, W, Hb):
    Ci, HW = x.shape
    Co = w.shape[3]
    N = Hb * W
    nb = HW // N
    wc = w.transpose(1, 0, 2, 3).reshape(3, 3 * Ci, Co)
    if a is None:
        body = functools.partial(
            lambda x_ref, m_ref, w_ref, h_ref, st_ref, **kw: _conv_body(
                x_ref, m_ref, w_ref, None, None, h_ref, st_ref, **kw),
            HW=HW, W=W, N=N, Ci=Ci, Co=Co, slope=slope, nb=nb)
        extra_in, extra_args = [], []
    else:
        body = functools.partial(_conv_body, HW=HW, W=W, N=N, Ci=Ci, Co=Co,
                                 slope=slope, nb=nb)
        extra_in = [
            pl.BlockSpec((Ci, 1), lambda i: (0, 0)),
            pl.BlockSpec((Ci, 1), lambda i: (0, 0)),
        ]
        extra_args = [a.reshape(Ci, 1), c.reshape(Ci, 1)]
    return pl.pallas_call(
        body,
        grid=(nb,),
        in_specs=[
            pl.BlockSpec((Ci, HW), lambda i: (0, 0)),
            pl.BlockSpec((1, HW), lambda i: (0, 0)),
            pl.BlockSpec((3, 3 * Ci, Co), lambda i: (0, 0, 0)),
        ] + extra_in,
        out_specs=[
            pl.BlockSpec((Co, N), lambda i: (0, i)),
            pl.BlockSpec((Co, 4), lambda i: (0, 0)),
        ],
        out_shape=[
            jax.ShapeDtypeStruct((Co, HW), jnp.float32),
            jax.ShapeDtypeStruct((Co, 4), jnp.float32),
        ],
    )(x, m, wc, *extra_args)


def _sel_mats(W):
    i0 = lax.broadcasted_iota(jnp.int32, (W, W // 2), 0)
    i1 = lax.broadcasted_iota(jnp.int32, (W, W // 2), 1)
    ev = (i0 == 2 * i1).astype(jnp.float32)
    od = (i0 == 2 * i1 + 1).astype(jnp.float32)
    return ev, od


def _lane_pairmax(r, W):
    ev, od = _sel_mats(W)
    sh = r.shape
    r2 = r.reshape(-1, W)
    re = lax.dot_general(r2, ev, (((1,), (0,)), ((), ())),
                         preferred_element_type=jnp.float32)
    ro = lax.dot_general(r2, od, (((1,), (0,)), ((), ())),
                         preferred_element_type=jnp.float32)
    return jnp.maximum(re, ro).reshape(sh[:-1] + (W // 2,))


def _fin_vals(h3, h1, m, a1, c1, a3, c3):
    res = _lrelu((h1 * a1 + c1) * m, 0.333)
    hin = (h3 * a3 + c3) * m
    return _lrelu(hin + res, 0.01)


def _fin_body(h3_ref, h1_ref, m_ref, a1_ref, c1_ref, a3_ref, c3_ref, y_ref):
    m = _binm(m_ref[...])
    y_ref[...] = _fin_vals(h3_ref[...], h1_ref[...], m, a1_ref[...],
                           c1_ref[...], a3_ref[...], c3_ref[...])


def _finpool_body(h3_ref, h1_ref, m_ref, a1_ref, c1_ref, a3_ref, c3_ref,
                  y_ref, yp_ref, mp_ref, *, Co, Hb, W):
    m = _binm(m_ref[...])
    y = _fin_vals(h3_ref[...], h1_ref[...], m[None],
                  a1_ref[...].reshape(Co, 1, 1), c1_ref[...].reshape(Co, 1, 1),
                  a3_ref[...].reshape(Co, 1, 1), c3_ref[...].reshape(Co, 1, 1))
    y_ref[...] = y
    neg = jnp.where(m[None] > 0, y, -1e30)
    n4 = neg.reshape(Co, Hb // 2, 2, W)
    r = jnp.maximum(n4[:, :, 0, :], n4[:, :, 1, :])
    p = _lane_pairmax(r, W)
    m4 = m.reshape(Hb // 2, 2, W)
    mr = jnp.maximum(m4[:, 0, :], m4[:, 1, :])
    m2 = _lane_pairmax(mr, W)
    yp_ref[...] = jnp.where(m2[None] > 0, p, 0.0)
    mp_ref[...] = m2


def _finpred_body(h3_ref, h1_ref, m_ref, a1_ref, c1_ref, a3_ref, c3_ref,
                  A_ref, y_ref, st_ref):
    i = pl.program_id(0)
    m = _binm(m_ref[...])
    y = _fin_vals(h3_ref[...], h1_ref[...], m, a1_ref[...], c1_ref[...],
                  a3_ref[...], c3_ref[...])
    y_ref[...] = y
    h = lax.dot_general(A_ref[...], y, (((1,), (0,)), ((), ())),
                        preferred_element_type=jnp.float32)
    hm = h * m
    s = hm.sum(1, keepdims=True)
    ss = (hm * hm).sum(1, keepdims=True)
    upd = jnp.concatenate([s, ss], axis=1)

    @pl.when(i == 0)
    def _():
        st_ref[...] = jnp.zeros_like(st_ref)

    st_ref[...] = st_ref[...] + upd


def _fin_specs_flat(Co, HW, NB):
    return [
        pl.BlockSpec((Co, NB), lambda i: (0, i)),
        pl.BlockSpec((Co, NB), lambda i: (0, i)),
        pl.BlockSpec((1, NB), lambda i: (0, i)),
        pl.BlockSpec((Co, 1), lambda i: (0, 0)),
        pl.BlockSpec((Co, 1), lambda i: (0, 0)),
        pl.BlockSpec((Co, 1), lambda i: (0, 0)),
        pl.BlockSpec((Co, 1), lambda i: (0, 0)),
    ]


def _finalize(h3, h1, m, a1, c1, a3, c3, NB):
    Co, HW = h3.shape
    nb = HW // NB
    return pl.pallas_call(
        _fin_body,
        grid=(nb,),
        in_specs=_fin_specs_flat(Co, HW, NB),
        out_specs=pl.BlockSpec((Co, NB), lambda i: (0, i)),
        out_shape=jax.ShapeDtypeStruct((Co, HW), jnp.float32),
    )(h3, h1, m, a1.reshape(Co, 1), c1.reshape(Co, 1), a3.reshape(Co, 1),
      c3.reshape(Co, 1))


def _finalize_pool(h3, h1, m, a1, c1, a3, c3, Hb):
    Co, H, W = h3.shape
    nb = H // Hb
    body = functools.partial(_finpool_body, Co=Co, Hb=Hb, W=W)
    return pl.pallas_call(
        body,
        grid=(nb,),
        in_specs=[
            pl.BlockSpec((Co, Hb, W), lambda i: (0, i, 0)),
            pl.BlockSpec((Co, Hb, W), lambda i: (0, i, 0)),
            pl.BlockSpec((Hb, W), lambda i: (i, 0)),
            pl.BlockSpec((Co, 1), lambda i: (0, 0)),
            pl.BlockSpec((Co, 1), lambda i: (0, 0)),
            pl.BlockSpec((Co, 1), lambda i: (0, 0)),
            pl.BlockSpec((Co, 1), lambda i: (0, 0)),
        ],
        out_specs=[
            pl.BlockSpec((Co, Hb, W), lambda i: (0, i, 0)),
            pl.BlockSpec((Co, Hb // 2, W // 2), lambda i: (0, i, 0)),
            pl.BlockSpec((Hb // 2, W // 2), lambda i: (i, 0)),
        ],
        out_shape=[
            jax.ShapeDtypeStruct((Co, H, W), jnp.float32),
            jax.ShapeDtypeStruct((Co, H // 2, W // 2), jnp.float32),
            jax.ShapeDtypeStruct((H // 2, W // 2), jnp.float32),
        ],
    )(h3, h1, m, a1.reshape(Co, 1), c1.reshape(Co, 1), a3.reshape(Co, 1),
      c3.reshape(Co, 1))


def _finalize_pred(h3, h1, m, a1, c1, a3, c3, A, NB):
    Co, HW = h3.shape
    D = A.shape[0]
    nb = HW // NB
    return pl.pallas_call(
        _finpred_body,
        grid=(nb,),
        in_specs=_fin_specs_flat(Co, HW, NB) + [
            pl.BlockSpec((D, Co), lambda i: (0, 0)),
        ],
        out_specs=[
            pl.BlockSpec((Co, NB), lambda i: (0, i)),
            pl.BlockSpec((D, 2), lambda i: (0, 0)),
        ],
        out_shape=[
            jax.ShapeDtypeStruct((Co, HW), jnp.float32),
            jax.ShapeDtypeStruct((D, 2), jnp.float32),
        ],
    )(h3, h1, m, a1.reshape(Co, 1), c1.reshape(Co, 1), a3.reshape(Co, 1),
      c3.reshape(Co, 1), A)


def _up_body(x_ref, s_ref, m_ref, w_ref, y_ref, dm_ref, *, Ci, D, Hb, W):
    i0 = lax.broadcasted_iota(jnp.int32, (W, 2 * W), 0)
    i1 = lax.broadcasted_iota(jnp.int32, (W, 2 * W), 1)
    p0 = (i1 == 2 * i0).astype(jnp.float32)
    p1 = (i1 == 2 * i0 + 1).astype(jnp.float32)

    x2 = x_ref[...].reshape(Ci, Hb * W)
    ys = []
    for ai in range(2):
        row = []
        for bi in range(2):
            v = lax.dot_general(
                w_ref[:, :, ai, bi], x2, (((0,), (0,)), ((), ())),
                preferred_element_type=jnp.float32)
            row.append(v)
        zr = (lax.dot_general(row[0].reshape(D * Hb, W), p0,
                              (((1,), (0,)), ((), ())),
                              preferred_element_type=jnp.float32) +
              lax.dot_general(row[1].reshape(D * Hb, W), p1,
                              (((1,), (0,)), ((), ())),
                              preferred_element_type=jnp.float32))
        ys.append(zr.reshape(D, Hb, 2 * W))
    z = jnp.stack(ys, axis=2).reshape(D, 2 * Hb, 2 * W)
    m2 = m_ref[...]
    mc = lax.dot_general(m2, p0 + p1, (((1,), (0,)), ((), ())),
                         preferred_element_type=jnp.float32)
    dm = jnp.stack([mc, mc], axis=1).reshape(2 * Hb, 2 * W)
    y_ref[...] = (z + s_ref[...]) * dm[None]
    dm_ref[...] = dm


def _up2(x, skip, m, w, Hb):
    Ci, H, W = x.shape
    D = w.shape[1]
    nb = H // Hb
    body = functools.partial(_up_body, Ci=Ci, D=D, Hb=Hb, W=W)
    return pl.pallas_call(
        body,
        grid=(nb,),
        in_specs=[
            pl.BlockSpec((Ci, Hb, W), lambda i: (0, i, 0)),
            pl.BlockSpec((D, 2 * Hb, 2 * W), lambda i: (0, i, 0)),
            pl.BlockSpec((Hb, W), lambda i: (i, 0)),
            pl.BlockSpec((Ci, D, 2, 2), lambda i: (0, 0, 0, 0)),
        ],
        out_specs=[
            pl.BlockSpec((D, 2 * Hb, 2 * W), lambda i: (0, i, 0)),
            pl.BlockSpec((2 * Hb, 2 * W), lambda i: (i, 0)),
        ],
        out_shape=[
            jax.ShapeDtypeStruct((D, 2 * H, 2 * W), jnp.float32),
            jax.ShapeDtypeStruct((2 * H, 2 * W), jnp.float32),
        ],
    )(x, skip, m, w)


def _pred2_body(x_ref, m_ref, a_ref, b_ref, aa_ref, cc_ref, o_ref):
    h = lax.dot_general(a_ref[...], x_ref[...], (((1,), (0,)), ((), ())),
                        preferred_element_type=jnp.float32)
    m = _binm(m_ref[...])
    q = (h * aa_ref[...] + cc_ref[...]) * m
    ql = _lrelu(q, 0.01) * m
    o = lax.dot_general(b_ref[...], ql, (((1,), (0,)), ((), ())),
                        preferred_element_type=jnp.float32)
    o0 = jnp.where(m > 0, o[0:1], 1.0)
    o1 = jnp.where(m > 0, o[1:2], 0.0)
    o_ref[...] = jnp.concatenate([o0, o1], axis=0)


def _pred2(x, m, A, B, a, c, NB):
    Ci, HW = x.shape
    D = A.shape[0]
    nb = HW // NB
    return pl.pallas_call(
        _pred2_body,
        grid=(nb,),
        in_specs=[
            pl.BlockSpec((Ci, NB), lambda i: (0, i)),
            pl.BlockSpec((1, NB), lambda i: (0, i)),
            pl.BlockSpec((D, Ci), lambda i: (0, 0)),
            pl.BlockSpec((2, D), lambda i: (0, 0)),
            pl.BlockSpec((D, 1), lambda i: (0, 0)),
            pl.BlockSpec((D, 1), lambda i: (0, 0)),
        ],
        out_specs=pl.BlockSpec((2, NB), lambda i: (0, i)),
        out_shape=jax.ShapeDtypeStruct((2, HW), jnp.float32),
    )(x, m, A, B, a.reshape(D, 1), c.reshape(D, 1))


def _scatter(coords, features):
    r, c = coords[:, 0], coords[:, 1]
    n = features.shape[0]
    aug = jnp.concatenate(
        [features, jnp.ones((n, 1), jnp.float32),
         jnp.zeros((n, 15), jnp.float32)], axis=1)
    out = jnp.zeros((_GRID, _GRID, 32), jnp.float32).at[r, c].add(aug)
    x = out[:, :, :_LAT].transpose(2, 0, 1)
    m0 = out[:, :, _LAT]
    return x, m0


def _bn_consts(st, g, b, n, eps):
    mu = st[:, 0] / n
    var = st[:, 1] / n - mu * mu
    inv = lax.rsqrt(var + eps)
    a = inv * g
    c = b - mu * a
    return a, c


def _resblock_convs(x, m, W1, g1, b1, W2, g2, b2, W3, W, Hb):
    h1, st1 = _conv3x3_flat(x, m, W1, None, None, 1.0, W, Hb)
    n = jnp.maximum(st1[0, 2], 1.0)
    a1, c1 = _bn_consts(st1, g1, b1, n, 1e-4)
    h2, st2 = _conv3x3_flat(h1, m, W2, a1, c1, 0.333, W, Hb)
    a2, c2 = _bn_consts(st2, g2, b2, n, 1e-4)
    h3, st3 = _conv3x3_flat(h2, m, W3, a2, c2, 0.333, W, Hb)
    mu3 = st3[:, 0] / n
    var3 = st3[:, 1] / n - mu3 * mu3
    inv3 = lax.rsqrt(var3 + 1e-5)
    return h3, h1, a1, c1, inv3, -mu3 * inv3, n


def kernel(coords, features, e0_W1, e0_g1, e0_b1, e0_W2, e0_g2, e0_b2, e0_W3,
           e1_W1, e1_g1, e1_b1, e1_W2, e1_g2, e1_b2, e1_W3,
           e2_W1, e2_g1, e2_b1, e2_W2, e2_g2, e2_b2, e2_W3,
           d0_up, d0_W1, d0_g1, d0_b1, d0_W2, d0_g2, d0_b2, d0_W3,
           d1_up, d1_W1, d1_g1, d1_b1, d1_W2, d1_g2, d1_b2, d1_W3,
           p0_A, p0_B, p1_A, p1_B):
    x, m0 = _scatter(coords, features)

    h3, h1, a1, c1, a3, c3, _ = _resblock_convs(
        x.reshape(16, 512 * 512), m0.reshape(1, 512 * 512),
        e0_W1, e0_g1, e0_b1, e0_W2, e0_g2, e0_b2, e0_W3, 512, 128)
    e0, xp, m1 = _finalize_pool(h3.reshape(16, 512, 512),
                                h1.reshape(16, 512, 512), m0,
                                a1, c1, a3, c3, 64)

    h3, h1, a1, c1, a3, c3, _ = _resblock_convs(
        xp.reshape(16, 256 * 256), m1.reshape(1, 256 * 256),
        e1_W1, e1_g1, e1_b1, e1_W2, e1_g2, e1_b2, e1_W3, 256, 128)
    e1, xp, m2 = _finalize_pool(h3.reshape(32, 256, 256),
                                h1.reshape(32, 256, 256), m1,
                                a1, c1, a3, c3, 64)

    m2f = m2.reshape(1, 128 * 128)
    h3, h1, a1, c1, a3, c3, _ = _resblock_convs(
        xp.reshape(32, 128 * 128), m2f,
        e2_W1, e2_g1, e2_b1, e2_W2, e2_g2, e2_b2, e2_W3, 128, 128)
    e2 = _finalize(h3, h1, m2f, a1, c1, a3, c3, 64 * 128).reshape(64, 128, 128)

    xd0, dm0 = _up2(e2, e1, m2, d0_up, 32)
    dm0f = dm0.reshape(1, 256 * 256)
    h3, h1, a1, c1, a3, c3, n_dm0 = _resblock_convs(
        xd0.reshape(32, 256 * 256), dm0f,
        d0_W1, d0_g1, d0_b1, d0_W2, d0_g2, d0_b2, d0_W3, 256, 128)
    d0f, pst0 = _finalize_pred(h3, h1, dm0f, a1, c1, a3, c3, p0_A, 64 * 256)
    mu = pst0[:, 0] / n_dm0
    var = pst0[:, 1] / n_dm0 - mu * mu
    inv = lax.rsqrt(var + 1e-5)
    log0 = _pred2(d0f, dm0f, p0_A, p0_B, inv, -mu * inv,
                  64 * 256).reshape(2, 256, 256)
    d0 = d0f.reshape(32, 256, 256)

    xd1, dm1 = _up2(d0, e0, dm0, d1_up, 32)
    dm1f = dm1.reshape(1, 512 * 512)
    h3, h1, a1, c1, a3, c3, n_dm1 = _resblock_convs(
        xd1.reshape(16, 512 * 512), dm1f,
        d1_W1, d1_g1, d1_b1, d1_W2, d1_g2, d1_b2, d1_W3, 512, 128)
    d1f, pst1 = _finalize_pred(h3, h1, dm1f, a1, c1, a3, c3, p1_A, 64 * 512)
    mu = pst1[:, 0] / n_dm1
    var = pst1[:, 1] / n_dm1 - mu * mu
    inv = lax.rsqrt(var + 1e-5)
    log1 = _pred2(d1f, dm1f, p1_A, p1_B, inv, -mu * inv,
                  64 * 512).reshape(2, 512, 512)
    d1 = d1f.reshape(16, 512, 512)

    return (d1, log1, log0, e0, e1, e2, d1, d0)


# speedup vs baseline: 1.0975x; 1.0975x over previous
import functools

import jax
import jax.numpy as jnp
from jax import lax
from jax.experimental import pallas as pl

_GRID = 512
_LAT = 16


def _lrelu(x, s):
    return jnp.maximum(x, s * x)


def _binm(m):
    return (m > 0).astype(jnp.float32)


def _conv_body(x_ref, m_ref, w_ref, a_ref, c_ref, h_ref, st_ref, *, HW, W, N,
               Ci, Co, slope, nb):
    i = pl.program_id(0)
    s0 = i * N
    lidx = lax.broadcasted_iota(jnp.int32, (1, N), 1)

    Hb = N // W
    H = nb * Hb
    rtop = jnp.maximum(i * Hb - 1, 0) * W
    rbot = jnp.minimum(i * Hb + Hb, H - 1) * W
    top = x_ref[:, pl.ds(rtop, W)]
    top = jnp.where(i == 0, 0.0, top)
    midw = x_ref[:, pl.ds(s0, N)]
    bot = x_ref[:, pl.ds(rbot, W)]
    bot = jnp.where(i == nb - 1, 0.0, bot)
    xb = jnp.concatenate([top, midw, bot], axis=1)

    if a_ref is None:
        t = xb
        mc = _binm(m_ref[:, pl.ds(s0, N)])
    else:
        mtop = m_ref[:, pl.ds(rtop, W)]
        mtop = jnp.where(i == 0, 0.0, mtop)
        mmid = m_ref[:, pl.ds(s0, N)]
        mbot = m_ref[:, pl.ds(rbot, W)]
        mbot = jnp.where(i == nb - 1, 0.0, mbot)
        mb = _binm(jnp.concatenate([mtop, mmid, mbot], axis=1))
        av = a_ref[...]
        cv = c_ref[...]
        t = _lrelu((xb * av + cv) * mb, slope)
        mc = mb[:, W:N + W]
    xs = jnp.concatenate(
        [t[:, 0:N], t[:, W:N + W], t[:, 2 * W:N + 2 * W]], axis=0)

    accs = [
        lax.dot_general(w_ref[dc], xs, (((0,), (0,)), ((), ())),
                        preferred_element_type=jnp.float32)
        for dc in range(3)
    ]
    a0 = jnp.pad(accs[0], ((0, 0), (1, 0)))[:, :N]
    a0 = jnp.where(lidx % W == 0, 0.0, a0)
    a2 = jnp.pad(accs[2], ((0, 0), (0, 1)))[:, 1:]
    a2 = jnp.where(lidx % W == W - 1, 0.0, a2)
    h = (a0 + accs[1] + a2) * mc
    h_ref[...] = h

    s = h.sum(1, keepdims=True)
    ss = (h * h).sum(1, keepdims=True)
    mn = mc.sum(1, keepdims=True)
    upd = jnp.concatenate(
        [s, ss, jnp.broadcast_to(mn, (Co, 1)), jnp.zeros((Co, 1), jnp.float32)],
        axis=1)

    @pl.when(i == 0)
    def _():
        st_ref[...] = jnp.zeros_like(st_ref)

    st_ref[...] = st_ref[...] + upd


def _conv3x3_flat(x, m, w, a, c, slope, W, Hb):
    Ci, HW = x.shape
    Co = w.shape[3]
    N = Hb * W
    nb = HW // N
    wc = w.transpose(1, 0, 2, 3).reshape(3, 3 * Ci, Co)
    if a is None:
        body = functools.partial(
            lambda x_ref, m_ref, w_ref, h_ref, st_ref, **kw: _conv_body(
                x_ref, m_ref, w_ref, None, None, h_ref, st_ref, **kw),
            HW=HW, W=W, N=N, Ci=Ci, Co=Co, slope=slope, nb=nb)
        extra_in, extra_args = [], []
    else:
        body = functools.partial(_conv_body, HW=HW, W=W, N=N, Ci=Ci, Co=Co,
                                 slope=slope, nb=nb)
        extra_in = [
            pl.BlockSpec((Ci, 1), lambda i: (0, 0)),
            pl.BlockSpec((Ci, 1), lambda i: (0, 0)),
        ]
        extra_args = [a.reshape(Ci, 1), c.reshape(Ci, 1)]
    return pl.pallas_call(
        body,
        grid=(nb,),
        in_specs=[
            pl.BlockSpec((Ci, HW), lambda i: (0, 0)),
            pl.BlockSpec((1, HW), lambda i: (0, 0)),
            pl.BlockSpec((3, 3 * Ci, Co), lambda i: (0, 0, 0)),
        ] + extra_in,
        out_specs=[
            pl.BlockSpec((Co, N), lambda i: (0, i)),
            pl.BlockSpec((Co, 4), lambda i: (0, 0)),
        ],
        out_shape=[
            jax.ShapeDtypeStruct((Co, HW), jnp.float32),
            jax.ShapeDtypeStruct((Co, 4), jnp.float32),
        ],
    )(x, m, wc, *extra_args)


def _sel_mats(W):
    i0 = lax.broadcasted_iota(jnp.int32, (W, W // 2), 0)
    i1 = lax.broadcasted_iota(jnp.int32, (W, W // 2), 1)
    ev = (i0 == 2 * i1).astype(jnp.float32)
    od = (i0 == 2 * i1 + 1).astype(jnp.float32)
    return ev, od


def _lane_pairmax(r, W):
    ev, od = _sel_mats(W)
    sh = r.shape
    r2 = r.reshape(-1, W)
    re = lax.dot_general(r2, ev, (((1,), (0,)), ((), ())),
                         preferred_element_type=jnp.float32)
    ro = lax.dot_general(r2, od, (((1,), (0,)), ((), ())),
                         preferred_element_type=jnp.float32)
    return jnp.maximum(re, ro).reshape(sh[:-1] + (W // 2,))


def _fin_vals(h3, h1, m, a1, c1, a3, c3):
    res = _lrelu((h1 * a1 + c1) * m, 0.333)
    hin = (h3 * a3 + c3) * m
    return _lrelu(hin + res, 0.01)


def _fin_body(h3_ref, h1_ref, m_ref, a1_ref, c1_ref, a3_ref, c3_ref, y_ref):
    m = _binm(m_ref[...])
    y_ref[...] = _fin_vals(h3_ref[...], h1_ref[...], m, a1_ref[...],
                           c1_ref[...], a3_ref[...], c3_ref[...])


def _finpool_body(h3_ref, h1_ref, m_ref, a1_ref, c1_ref, a3_ref, c3_ref,
                  y_ref, yp_ref, mp_ref, *, Co, Hb, W):
    m = _binm(m_ref[...])
    y = _fin_vals(h3_ref[...], h1_ref[...], m[None],
                  a1_ref[...].reshape(Co, 1, 1), c1_ref[...].reshape(Co, 1, 1),
                  a3_ref[...].reshape(Co, 1, 1), c3_ref[...].reshape(Co, 1, 1))
    y_ref[...] = y
    neg = jnp.where(m[None] > 0, y, -1e30)
    n4 = neg.reshape(Co, Hb // 2, 2, W)
    r = jnp.maximum(n4[:, :, 0, :], n4[:, :, 1, :])
    p = _lane_pairmax(r, W)
    m4 = m.reshape(Hb // 2, 2, W)
    mr = jnp.maximum(m4[:, 0, :], m4[:, 1, :])
    m2 = _lane_pairmax(mr, W)
    yp_ref[...] = jnp.where(m2[None] > 0, p, 0.0)
    mp_ref[...] = m2


def _finpred_body(h3_ref, h1_ref, m_ref, a1_ref, c1_ref, a3_ref, c3_ref,
                  A_ref, y_ref, st_ref):
    i = pl.program_id(0)
    m = _binm(m_ref[...])
    y = _fin_vals(h3_ref[...], h1_ref[...], m, a1_ref[...], c1_ref[...],
                  a3_ref[...], c3_ref[...])
    y_ref[...] = y
    h = lax.dot_general(A_ref[...], y, (((1,), (0,)), ((), ())),
                        preferred_element_type=jnp.float32)
    hm = h * m
    s = hm.sum(1, keepdims=True)
    ss = (hm * hm).sum(1, keepdims=True)
    upd = jnp.concatenate([s, ss], axis=1)

    @pl.when(i == 0)
    def _():
        st_ref[...] = jnp.zeros_like(st_ref)

    st_ref[...] = st_ref[...] + upd


def _fin_specs_flat(Co, HW, NB):
    return [
        pl.BlockSpec((Co, NB), lambda i: (0, i)),
        pl.BlockSpec((Co, NB), lambda i: (0, i)),
        pl.BlockSpec((1, NB), lambda i: (0, i)),
        pl.BlockSpec((Co, 1), lambda i: (0, 0)),
        pl.BlockSpec((Co, 1), lambda i: (0, 0)),
        pl.BlockSpec((Co, 1), lambda i: (0, 0)),
        pl.BlockSpec((Co, 1), lambda i: (0, 0)),
    ]


def _finalize(h3, h1, m, a1, c1, a3, c3, NB):
    Co, HW = h3.shape
    nb = HW // NB
    return pl.pallas_call(
        _fin_body,
        grid=(nb,),
        in_specs=_fin_specs_flat(Co, HW, NB),
        out_specs=pl.BlockSpec((Co, NB), lambda i: (0, i)),
        out_shape=jax.ShapeDtypeStruct((Co, HW), jnp.float32),
    )(h3, h1, m, a1.reshape(Co, 1), c1.reshape(Co, 1), a3.reshape(Co, 1),
      c3.reshape(Co, 1))


def _finalize_pool(h3, h1, m, a1, c1, a3, c3, Hb):
    Co, H, W = h3.shape
    nb = H // Hb
    body = functools.partial(_finpool_body, Co=Co, Hb=Hb, W=W)
    return pl.pallas_call(
        body,
        grid=(nb,),
        in_specs=[
            pl.BlockSpec((Co, Hb, W), lambda i: (0, i, 0)),
            pl.BlockSpec((Co, Hb, W), lambda i: (0, i, 0)),
            pl.BlockSpec((Hb, W), lambda i: (i, 0)),
            pl.BlockSpec((Co, 1), lambda i: (0, 0)),
            pl.BlockSpec((Co, 1), lambda i: (0, 0)),
            pl.BlockSpec((Co, 1), lambda i: (0, 0)),
            pl.BlockSpec((Co, 1), lambda i: (0, 0)),
        ],
        out_specs=[
            pl.BlockSpec((Co, Hb, W), lambda i: (0, i, 0)),
            pl.BlockSpec((Co, Hb // 2, W // 2), lambda i: (0, i, 0)),
            pl.BlockSpec((Hb // 2, W // 2), lambda i: (i, 0)),
        ],
        out_shape=[
            jax.ShapeDtypeStruct((Co, H, W), jnp.float32),
            jax.ShapeDtypeStruct((Co, H // 2, W // 2), jnp.float32),
            jax.ShapeDtypeStruct((H // 2, W // 2), jnp.float32),
        ],
    )(h3, h1, m, a1.reshape(Co, 1), c1.reshape(Co, 1), a3.reshape(Co, 1),
      c3.reshape(Co, 1))


def _finalize_pred(h3, h1, m, a1, c1, a3, c3, A, NB):
    Co, HW = h3.shape
    D = A.shape[0]
    nb = HW // NB
    return pl.pallas_call(
        _finpred_body,
        grid=(nb,),
        in_specs=_fin_specs_flat(Co, HW, NB) + [
            pl.BlockSpec((D, Co), lambda i: (0, 0)),
        ],
        out_specs=[
            pl.BlockSpec((Co, NB), lambda i: (0, i)),
            pl.BlockSpec((D, 2), lambda i: (0, 0)),
        ],
        out_shape=[
            jax.ShapeDtypeStruct((Co, HW), jnp.float32),
            jax.ShapeDtypeStruct((D, 2), jnp.float32),
        ],
    )(h3, h1, m, a1.reshape(Co, 1), c1.reshape(Co, 1), a3.reshape(Co, 1),
      c3.reshape(Co, 1), A)


def _up_body(x_ref, s_ref, m_ref, w_ref, y_ref, dm_ref, *, Ci, D, Hb, W):
    i0 = lax.broadcasted_iota(jnp.int32, (W, 2 * W), 0)
    i1 = lax.broadcasted_iota(jnp.int32, (W, 2 * W), 1)
    p0 = (i1 == 2 * i0).astype(jnp.float32)
    p1 = (i1 == 2 * i0 + 1).astype(jnp.float32)

    x2 = x_ref[...].reshape(Ci, Hb * W)
    ys = []
    for ai in range(2):
        row = []
        for bi in range(2):
            v = lax.dot_general(
                w_ref[:, :, ai, bi], x2, (((0,), (0,)), ((), ())),
                preferred_element_type=jnp.float32)
            row.append(v)
        zr = (lax.dot_general(row[0].reshape(D * Hb, W), p0,
                              (((1,), (0,)), ((), ())),
                              preferred_element_type=jnp.float32) +
              lax.dot_general(row[1].reshape(D * Hb, W), p1,
                              (((1,), (0,)), ((), ())),
                              preferred_element_type=jnp.float32))
        ys.append(zr.reshape(D, Hb, 2 * W))
    z = jnp.stack(ys, axis=2).reshape(D, 2 * Hb, 2 * W)
    m2 = m_ref[...]
    mc = lax.dot_general(m2, p0 + p1, (((1,), (0,)), ((), ())),
                         preferred_element_type=jnp.float32)
    dm = jnp.stack([mc, mc], axis=1).reshape(2 * Hb, 2 * W)
    y_ref[...] = (z + s_ref[...]) * dm[None]
    dm_ref[...] = dm


def _up2(x, skip, m, w, Hb):
    Ci, H, W = x.shape
    D = w.shape[1]
    nb = H // Hb
    body = functools.partial(_up_body, Ci=Ci, D=D, Hb=Hb, W=W)
    return pl.pallas_call(
        body,
        grid=(nb,),
        in_specs=[
            pl.BlockSpec((Ci, Hb, W), lambda i: (0, i, 0)),
            pl.BlockSpec((D, 2 * Hb, 2 * W), lambda i: (0, i, 0)),
            pl.BlockSpec((Hb, W), lambda i: (i, 0)),
            pl.BlockSpec((Ci, D, 2, 2), lambda i: (0, 0, 0, 0)),
        ],
        out_specs=[
            pl.BlockSpec((D, 2 * Hb, 2 * W), lambda i: (0, i, 0)),
            pl.BlockSpec((2 * Hb, 2 * W), lambda i: (i, 0)),
        ],
        out_shape=[
            jax.ShapeDtypeStruct((D, 2 * H, 2 * W), jnp.float32),
            jax.ShapeDtypeStruct((2 * H, 2 * W), jnp.float32),
        ],
    )(x, skip, m, w)


def _pred2_body(x_ref, m_ref, a_ref, b_ref, aa_ref, cc_ref, o_ref):
    h = lax.dot_general(a_ref[...], x_ref[...], (((1,), (0,)), ((), ())),
                        preferred_element_type=jnp.float32)
    m = _binm(m_ref[...])
    q = (h * aa_ref[...] + cc_ref[...]) * m
    ql = _lrelu(q, 0.01) * m
    o = lax.dot_general(b_ref[...], ql, (((1,), (0,)), ((), ())),
                        preferred_element_type=jnp.float32)
    o0 = jnp.where(m > 0, o[0:1], 1.0)
    o1 = jnp.where(m > 0, o[1:2], 0.0)
    o_ref[...] = jnp.concatenate([o0, o1], axis=0)


def _pred2(x, m, A, B, a, c, NB):
    Ci, HW = x.shape
    D = A.shape[0]
    nb = HW // NB
    return pl.pallas_call(
        _pred2_body,
        grid=(nb,),
        in_specs=[
            pl.BlockSpec((Ci, NB), lambda i: (0, i)),
            pl.BlockSpec((1, NB), lambda i: (0, i)),
            pl.BlockSpec((D, Ci), lambda i: (0, 0)),
            pl.BlockSpec((2, D), lambda i: (0, 0)),
            pl.BlockSpec((D, 1), lambda i: (0, 0)),
            pl.BlockSpec((D, 1), lambda i: (0, 0)),
        ],
        out_specs=pl.BlockSpec((2, NB), lambda i: (0, i)),
        out_shape=jax.ShapeDtypeStruct((2, HW), jnp.float32),
    )(x, m, A, B, a.reshape(D, 1), c.reshape(D, 1))


def _scatter(coords, features):
    r, c = coords[:, 0], coords[:, 1]
    n = features.shape[0]
    aug = jnp.concatenate(
        [features, jnp.ones((n, 1), jnp.float32)], axis=1)
    out = jnp.zeros((_GRID, _GRID, _LAT + 1), jnp.float32).at[r, c].add(aug)
    x = out[:, :, :_LAT].transpose(2, 0, 1)
    m0 = out[:, :, _LAT]
    return x, m0


def _bn_consts(st, g, b, n, eps):
    mu = st[:, 0] / n
    var = st[:, 1] / n - mu * mu
    inv = lax.rsqrt(var + eps)
    a = inv * g
    c = b - mu * a
    return a, c


def _resblock_convs(x, m, W1, g1, b1, W2, g2, b2, W3, W, Hb):
    h1, st1 = _conv3x3_flat(x, m, W1, None, None, 1.0, W, Hb)
    n = jnp.maximum(st1[0, 2], 1.0)
    a1, c1 = _bn_consts(st1, g1, b1, n, 1e-4)
    h2, st2 = _conv3x3_flat(h1, m, W2, a1, c1, 0.333, W, Hb)
    a2, c2 = _bn_consts(st2, g2, b2, n, 1e-4)
    h3, st3 = _conv3x3_flat(h2, m, W3, a2, c2, 0.333, W, Hb)
    mu3 = st3[:, 0] / n
    var3 = st3[:, 1] / n - mu3 * mu3
    inv3 = lax.rsqrt(var3 + 1e-5)
    return h3, h1, a1, c1, inv3, -mu3 * inv3, n


def kernel(coords, features, e0_W1, e0_g1, e0_b1, e0_W2, e0_g2, e0_b2, e0_W3,
           e1_W1, e1_g1, e1_b1, e1_W2, e1_g2, e1_b2, e1_W3,
           e2_W1, e2_g1, e2_b1, e2_W2, e2_g2, e2_b2, e2_W3,
           d0_up, d0_W1, d0_g1, d0_b1, d0_W2, d0_g2, d0_b2, d0_W3,
           d1_up, d1_W1, d1_g1, d1_b1, d1_W2, d1_g2, d1_b2, d1_W3,
           p0_A, p0_B, p1_A, p1_B):
    x, m0 = _scatter(coords, features)

    h3, h1, a1, c1, a3, c3, _ = _resblock_convs(
        x.reshape(16, 512 * 512), m0.reshape(1, 512 * 512),
        e0_W1, e0_g1, e0_b1, e0_W2, e0_g2, e0_b2, e0_W3, 512, 128)
    e0, xp, m1 = _finalize_pool(h3.reshape(16, 512, 512),
                                h1.reshape(16, 512, 512), m0,
                                a1, c1, a3, c3, 64)

    h3, h1, a1, c1, a3, c3, _ = _resblock_convs(
        xp.reshape(16, 256 * 256), m1.reshape(1, 256 * 256),
        e1_W1, e1_g1, e1_b1, e1_W2, e1_g2, e1_b2, e1_W3, 256, 128)
    e1, xp, m2 = _finalize_pool(h3.reshape(32, 256, 256),
                                h1.reshape(32, 256, 256), m1,
                                a1, c1, a3, c3, 64)

    m2f = m2.reshape(1, 128 * 128)
    h3, h1, a1, c1, a3, c3, _ = _resblock_convs(
        xp.reshape(32, 128 * 128), m2f,
        e2_W1, e2_g1, e2_b1, e2_W2, e2_g2, e2_b2, e2_W3, 128, 128)
    e2 = _finalize(h3, h1, m2f, a1, c1, a3, c3, 64 * 128).reshape(64, 128, 128)

    xd0, dm0 = _up2(e2, e1, m2, d0_up, 32)
    dm0f = dm0.reshape(1, 256 * 256)
    h3, h1, a1, c1, a3, c3, n_dm0 = _resblock_convs(
        xd0.reshape(32, 256 * 256), dm0f,
        d0_W1, d0_g1, d0_b1, d0_W2, d0_g2, d0_b2, d0_W3, 256, 128)
    d0f, pst0 = _finalize_pred(h3, h1, dm0f, a1, c1, a3, c3, p0_A, 64 * 256)
    mu = pst0[:, 0] / n_dm0
    var = pst0[:, 1] / n_dm0 - mu * mu
    inv = lax.rsqrt(var + 1e-5)
    log0 = _pred2(d0f, dm0f, p0_A, p0_B, inv, -mu * inv,
                  64 * 256).reshape(2, 256, 256)
    d0 = d0f.reshape(32, 256, 256)

    xd1, dm1 = _up2(d0, e0, dm0, d1_up, 32)
    dm1f = dm1.reshape(1, 512 * 512)
    h3, h1, a1, c1, a3, c3, n_dm1 = _resblock_convs(
        xd1.reshape(16, 512 * 512), dm1f,
        d1_W1, d1_g1, d1_b1, d1_W2, d1_g2, d1_b2, d1_W3, 512, 128)
    d1f, pst1 = _finalize_pred(h3, h1, dm1f, a1, c1, a3, c3, p1_A, 64 * 512)
    mu = pst1[:, 0] / n_dm1
    var = pst1[:, 1] / n_dm1 - mu * mu
    inv = lax.rsqrt(var + 1e-5)
    log1 = _pred2(d1f, dm1f, p1_A, p1_B, inv, -mu * inv,
                  64 * 512).reshape(2, 512, 512)
    d1 = d1f.reshape(16, 512, 512)

    return (d1, log1, log0, e0, e1, e2, d1, d0)
